# baseline (speedup 1.0000x reference)
import functools

import jax
import jax.numpy as jnp
from jax import lax
from jax.experimental import pallas as pl
from jax.experimental.pallas import tpu as pltpu
from jax.experimental.pallas import tpu_sc as plsc

B = 16384
NUM_STAGES = 16
NUM_VIEWS = 8
NUM_COMBOS = NUM_STAGES * NUM_VIEWS
EMBED_DIM = 1024
HIDDEN_DIM = 4096
NUM_EXPERTS = 64
SEL_W = 16

H_BLK = 512
N_HBLKS = HIDDEN_DIM // H_BLK

SC_CORES = 2
SC_SUBCORES = 16
SC_LANES = 16
NW = SC_CORES * SC_SUBCORES
B_PER_W = B // NW
IDX_CHUNK = 128
N_CHUNKS = B_PER_W // IDX_CHUNK


def _router_table_body(st_ref, vt_ref, w1_ref, b1_ref, w2_ref, b2_ref,
                       out_ref, sel_ref, acc_ref):
    j = pl.program_id(0)
    st = st_ref[...]
    vt = vt_ref[...]
    zs = jnp.broadcast_to(st[:, None, :], (NUM_STAGES, NUM_VIEWS, EMBED_DIM))
    zs = zs.reshape(NUM_COMBOS, EMBED_DIM)
    zv = jnp.broadcast_to(vt[None, :, :], (NUM_STAGES, NUM_VIEWS, EMBED_DIM))
    zv = zv.reshape(NUM_COMBOS, EMBED_DIM)
    z = jnp.concatenate([zs, zv], axis=1)

    h = jnp.dot(z, w1_ref[...], preferred_element_type=jnp.float32)
    h = jnp.maximum(h + b1_ref[...], 0.0)
    part = jnp.dot(h, w2_ref[...], preferred_element_type=jnp.float32)

    @pl.when(j == 0)
    def _init():
        acc_ref[...] = part

    @pl.when(j > 0)
    def _accum():
        acc_ref[...] = acc_ref[...] + part

    @pl.when(j == N_HBLKS - 1)
    def _finish():
        logits = acc_ref[...] + b2_ref[...]
        m = jnp.max(logits, axis=1, keepdims=True)
        e = jnp.exp(logits - m)
        probs = e / jnp.sum(e, axis=1, keepdims=True)
        col = lax.broadcasted_iota(jnp.int32, (NUM_COMBOS, NUM_EXPERTS), 1)
        pmax = jnp.max(probs, axis=1, keepdims=True)
        sel = jnp.min(jnp.where(probs == pmax, col, NUM_EXPERTS), axis=1)
        out_ref[...] = probs
        sel_ref[...] = jnp.broadcast_to(sel[:, None], (NUM_COMBOS, SEL_W))


def _router_table(stage_table, view_table, W1, b1, W2, b2):
    return pl.pallas_call(
        _router_table_body,
        grid=(N_HBLKS,),
        in_specs=[
            pl.BlockSpec((NUM_STAGES, EMBED_DIM), lambda j: (0, 0)),
            pl.BlockSpec((NUM_VIEWS, EMBED_DIM), lambda j: (0, 0)),
            pl.BlockSpec((2 * EMBED_DIM, H_BLK), lambda j: (0, j)),
            pl.BlockSpec((1, H_BLK), lambda j: (0, j)),
            pl.BlockSpec((H_BLK, NUM_EXPERTS), lambda j: (j, 0)),
            pl.BlockSpec((1, NUM_EXPERTS), lambda j: (0, 0)),
        ],
        out_specs=[pl.BlockSpec((NUM_COMBOS, NUM_EXPERTS), lambda j: (0, 0)),
                   pl.BlockSpec((NUM_COMBOS, SEL_W), lambda j: (0, 0))],
        out_shape=[jax.ShapeDtypeStruct((NUM_COMBOS, NUM_EXPERTS), jnp.float32),
                   jax.ShapeDtypeStruct((NUM_COMBOS, SEL_W), jnp.int32)],
        scratch_shapes=[pltpu.VMEM((NUM_COMBOS, NUM_EXPERTS), jnp.float32)],
        compiler_params=pltpu.CompilerParams(
            dimension_semantics=("arbitrary",)),
    )(stage_table, view_table, W1, b1.reshape(1, HIDDEN_DIM), W2,
      b2.reshape(1, NUM_EXPERTS))


def _dispatch_body(tabp_hbm, tabs_hbm, sids_hbm, vids_hbm, outp_hbm,
                   outs_hbm, sid_v, vid_v, cidx_v, rowsp_v, rowss_v,
                   sem, semw):
    wid = lax.axis_index("s") * SC_CORES + lax.axis_index("c")
    base = wid * B_PER_W
    pltpu.sync_copy(sids_hbm.at[pl.ds(base, B_PER_W)], sid_v)
    pltpu.sync_copy(vids_hbm.at[pl.ds(base, B_PER_W)], vid_v)
    for g in range(N_CHUNKS):
        for k in range(IDX_CHUNK // SC_LANES):
            off = g * IDX_CHUNK + k * SC_LANES
            s = sid_v[pl.ds(off, SC_LANES)]
            v = vid_v[pl.ds(off, SC_LANES)]
            cidx_v[g, pl.ds(k * SC_LANES, SC_LANES)] = s * NUM_VIEWS + v
    gathers = []
    for g in range(N_CHUNKS):
        gathers.append(pltpu.async_copy(
            tabp_hbm.at[cidx_v.at[g]], rowsp_v.at[g], sem))
        gathers.append(pltpu.async_copy(
            tabs_hbm.at[cidx_v.at[g]], rowss_v.at[g], sem))
    writes = []
    for g in range(N_CHUNKS):
        gathers[2 * g].wait()
        writes.append(pltpu.async_copy(
            rowsp_v.at[g],
            outp_hbm.at[pl.ds(base + g * IDX_CHUNK, IDX_CHUNK)], semw))
        gathers[2 * g + 1].wait()
        writes.append(pltpu.async_copy(
            rowss_v.at[g],
            outs_hbm.at[pl.ds(base + g * IDX_CHUNK, IDX_CHUNK)], semw))
    for w in writes:
        w.wait()


def _dispatch(tableP, tableS, stage_ids, view_ids):
    mesh = plsc.VectorSubcoreMesh(core_axis_name="c", subcore_axis_name="s")
    run = functools.partial(
        pl.kernel,
        mesh=mesh,
        out_type=(
            jax.ShapeDtypeStruct((B, NUM_EXPERTS), jnp.float32),
            jax.ShapeDtypeStruct((B, SEL_W), jnp.int32),
        ),
        scratch_types=[
            pltpu.VMEM((B_PER_W,), jnp.int32),
            pltpu.VMEM((B_PER_W,), jnp.int32),
            pltpu.VMEM((N_CHUNKS, IDX_CHUNK), jnp.int32),
            pltpu.VMEM((N_CHUNKS, IDX_CHUNK, NUM_EXPERTS), jnp.float32),
            pltpu.VMEM((N_CHUNKS, IDX_CHUNK, SEL_W), jnp.int32),
            pltpu.SemaphoreType.DMA,
            pltpu.SemaphoreType.DMA,
        ],
        compiler_params=pltpu.CompilerParams(use_tc_tiling_on_sc=False),
    )(_dispatch_body)
    return run(tableP, tableS, stage_ids, view_ids)


def kernel(stage_ids, view_ids, stage_table, view_table, W1, b1, W2, b2):
    stage_ids = stage_ids.astype(jnp.int32)
    view_ids = view_ids.astype(jnp.int32)
    tableP, tableS = _router_table(stage_table, view_table, W1, b1, W2, b2)
    outP, outS = _dispatch(tableP, tableS, stage_ids, view_ids)
    return (outP, outS[:, 0])


# speedup vs baseline: 1.0943x; 1.0943x over previous
import functools

import jax
import jax.numpy as jnp
from jax import lax
from jax.experimental import pallas as pl
from jax.experimental.pallas import tpu as pltpu
from jax.experimental.pallas import tpu_sc as plsc

B = 16384
NUM_STAGES = 16
NUM_VIEWS = 8
NUM_COMBOS = NUM_STAGES * NUM_VIEWS
EMBED_DIM = 1024
HIDDEN_DIM = 4096
NUM_EXPERTS = 64
PACK_W = 128

H_BLK = 1024
N_HBLKS = HIDDEN_DIM // H_BLK

SC_CORES = 2
SC_SUBCORES = 16
SC_LANES = 16
NW = SC_CORES * SC_SUBCORES
B_PER_W = B // NW
IDX_CHUNK = 128
N_CHUNKS = B_PER_W // IDX_CHUNK


def _router_table_body(st_ref, vt_ref, w1_ref, b1_ref, w2_ref, b2_ref,
                       out_ref, acc_ref):
    j = pl.program_id(0)
    p1 = jnp.dot(st_ref[...], w1_ref[:EMBED_DIM, :],
                 preferred_element_type=jnp.float32)
    p2 = jnp.dot(vt_ref[...], w1_ref[EMBED_DIM:, :],
                 preferred_element_type=jnp.float32)
    row = lax.broadcasted_iota(jnp.int32, (NUM_COMBOS, 1), 0)
    ohs = (row // NUM_VIEWS ==
           lax.broadcasted_iota(jnp.int32, (NUM_COMBOS, NUM_STAGES), 1)
           ).astype(jnp.float32)
    ohv = (row % NUM_VIEWS ==
           lax.broadcasted_iota(jnp.int32, (NUM_COMBOS, NUM_VIEWS), 1)
           ).astype(jnp.float32)
    h = (jnp.dot(ohs, p1, preferred_element_type=jnp.float32)
         + jnp.dot(ohv, p2, preferred_element_type=jnp.float32))
    h = jnp.maximum(h + b1_ref[...][None, :], 0.0)
    part = jnp.dot(h, w2_ref[...], preferred_element_type=jnp.float32)

    @pl.when(j == 0)
    def _init():
        acc_ref[...] = part

    @pl.when(j > 0)
    def _accum():
        acc_ref[...] = acc_ref[...] + part

    @pl.when(j == N_HBLKS - 1)
    def _finish():
        logits = acc_ref[...] + b2_ref[...][None, :]
        m = jnp.max(logits, axis=1, keepdims=True)
        e = jnp.exp(logits - m)
        probs = e / jnp.sum(e, axis=1, keepdims=True)
        col = lax.broadcasted_iota(jnp.int32, (NUM_COMBOS, NUM_EXPERTS), 1)
        pmax = jnp.max(probs, axis=1, keepdims=True)
        sel = jnp.min(jnp.where(probs == pmax, col, NUM_EXPERTS), axis=1)
        self32 = sel.astype(jnp.float32)
        out_ref[...] = jnp.concatenate(
            [probs,
             jnp.broadcast_to(self32[:, None],
                              (NUM_COMBOS, PACK_W - NUM_EXPERTS))], axis=1)


def _router_table(stage_table, view_table, W1, b1, W2, b2):
    return pl.pallas_call(
        _router_table_body,
        grid=(N_HBLKS,),
        in_specs=[
            pl.BlockSpec((NUM_STAGES, EMBED_DIM), lambda j: (0, 0)),
            pl.BlockSpec((NUM_VIEWS, EMBED_DIM), lambda j: (0, 0)),
            pl.BlockSpec((2 * EMBED_DIM, H_BLK), lambda j: (0, j)),
            pl.BlockSpec((H_BLK,), lambda j: (j,)),
            pl.BlockSpec((H_BLK, NUM_EXPERTS), lambda j: (j, 0)),
            pl.BlockSpec((NUM_EXPERTS,), lambda j: (0,)),
        ],
        out_specs=pl.BlockSpec((NUM_COMBOS, PACK_W), lambda j: (0, 0)),
        out_shape=jax.ShapeDtypeStruct((NUM_COMBOS, PACK_W), jnp.float32),
        scratch_shapes=[pltpu.VMEM((NUM_COMBOS, NUM_EXPERTS), jnp.float32)],
        compiler_params=pltpu.CompilerParams(
            dimension_semantics=("arbitrary",)),
    )(stage_table, view_table, W1, b1, W2, b2)


def _dispatch_body(table_hbm, sids_hbm, vids_hbm, out_hbm,
                   sid_v, vid_v, cidx_v, rows_v, sem, semw):
    wid = lax.axis_index("s") * SC_CORES + lax.axis_index("c")
    base = wid * B_PER_W
    pltpu.sync_copy(sids_hbm.at[pl.ds(base, B_PER_W)], sid_v)
    pltpu.sync_copy(vids_hbm.at[pl.ds(base, B_PER_W)], vid_v)
    for g in range(N_CHUNKS):
        for k in range(IDX_CHUNK // SC_LANES):
            off = g * IDX_CHUNK + k * SC_LANES
            s = sid_v[pl.ds(off, SC_LANES)]
            v = vid_v[pl.ds(off, SC_LANES)]
            cidx_v[g, pl.ds(k * SC_LANES, SC_LANES)] = s * NUM_VIEWS + v
    gathers = [
        pltpu.async_copy(table_hbm.at[cidx_v.at[g]], rows_v.at[g], sem)
        for g in range(N_CHUNKS)
    ]
    writes = []
    for g in range(N_CHUNKS):
        gathers[g].wait()
        writes.append(pltpu.async_copy(
            rows_v.at[g],
            out_hbm.at[pl.ds(base + g * IDX_CHUNK, IDX_CHUNK)], semw))
    for w in writes:
        w.wait()


def _dispatch(table, stage_ids, view_ids):
    mesh = plsc.VectorSubcoreMesh(core_axis_name="c", subcore_axis_name="s")
    run = functools.partial(
        pl.kernel,
        mesh=mesh,
        out_type=jax.ShapeDtypeStruct((B, PACK_W), jnp.float32),
        scratch_types=[
            pltpu.VMEM((B_PER_W,), jnp.int32),
            pltpu.VMEM((B_PER_W,), jnp.int32),
            pltpu.VMEM((N_CHUNKS, IDX_CHUNK), jnp.int32),
            pltpu.VMEM((N_CHUNKS, IDX_CHUNK, PACK_W), jnp.float32),
            pltpu.SemaphoreType.DMA,
            pltpu.SemaphoreType.DMA,
        ],
    )(_dispatch_body)
    return run(table, stage_ids, view_ids)


def kernel(stage_ids, view_ids, stage_table, view_table, W1, b1, W2, b2):
    stage_ids = stage_ids.astype(jnp.int32)
    view_ids = view_ids.astype(jnp.int32)
    table = _router_table(stage_table, view_table, W1, b1, W2, b2)
    packed = _dispatch(table, stage_ids, view_ids)
    expert_probs = packed[:, :NUM_EXPERTS]
    selected_expert = packed[:, NUM_EXPERTS].astype(jnp.int32)
    return (expert_probs, selected_expert)


# speedup vs baseline: 1.0999x; 1.0051x over previous
import functools

import jax
import jax.numpy as jnp
from jax import lax
from jax.experimental import pallas as pl
from jax.experimental.pallas import tpu as pltpu
from jax.experimental.pallas import tpu_sc as plsc

B = 16384
NUM_STAGES = 16
NUM_VIEWS = 8
NUM_COMBOS = NUM_STAGES * NUM_VIEWS
EMBED_DIM = 1024
HIDDEN_DIM = 4096
NUM_EXPERTS = 64
PACK_W = 128

H_BLK = 1024
N_HBLKS = HIDDEN_DIM // H_BLK

SC_CORES = 2
SC_SUBCORES = 16
SC_LANES = 16
NW = SC_CORES * SC_SUBCORES
B_PER_W = B // NW
IDX_CHUNK = 128
N_CHUNKS = B_PER_W // IDX_CHUNK


def _router_table_body(st_ref, vt_ref, w1_ref, b1_ref, w2_ref, b2_ref,
                       out_ref, acc_ref):
    j = pl.program_id(0)
    p1 = jnp.dot(st_ref[...], w1_ref[:EMBED_DIM, :],
                 preferred_element_type=jnp.float32)
    p2 = jnp.dot(vt_ref[...], w1_ref[EMBED_DIM:, :],
                 preferred_element_type=jnp.float32)
    p1r = jnp.broadcast_to(p1[:, None, :],
                           (NUM_STAGES, NUM_VIEWS, H_BLK))
    p1r = p1r.reshape(NUM_COMBOS, H_BLK)
    p2r = jnp.broadcast_to(p2[None, :, :],
                           (NUM_STAGES, NUM_VIEWS, H_BLK))
    p2r = p2r.reshape(NUM_COMBOS, H_BLK)
    h = jnp.maximum(p1r + p2r + b1_ref[...][None, :], 0.0)
    part = jnp.dot(h, w2_ref[...], preferred_element_type=jnp.float32)

    @pl.when(j == 0)
    def _init():
        acc_ref[...] = part

    @pl.when(j > 0)
    def _accum():
        acc_ref[...] = acc_ref[...] + part

    @pl.when(j == N_HBLKS - 1)
    def _finish():
        logits = acc_ref[...] + b2_ref[...][None, :]
        m = jnp.max(logits, axis=1, keepdims=True)
        e = jnp.exp(logits - m)
        probs = e / jnp.sum(e, axis=1, keepdims=True)
        col = lax.broadcasted_iota(jnp.int32, (NUM_COMBOS, NUM_EXPERTS), 1)
        pmax = jnp.max(probs, axis=1, keepdims=True)
        sel = jnp.min(jnp.where(probs == pmax, col, NUM_EXPERTS), axis=1)
        self32 = sel.astype(jnp.float32)
        out_ref[...] = jnp.concatenate(
            [probs,
             jnp.broadcast_to(self32[:, None],
                              (NUM_COMBOS, PACK_W - NUM_EXPERTS))], axis=1)


def _router_table(stage_table, view_table, W1, b1, W2, b2):
    return pl.pallas_call(
        _router_table_body,
        grid=(N_HBLKS,),
        in_specs=[
            pl.BlockSpec((NUM_STAGES, EMBED_DIM), lambda j: (0, 0)),
            pl.BlockSpec((NUM_VIEWS, EMBED_DIM), lambda j: (0, 0)),
            pl.BlockSpec((2 * EMBED_DIM, H_BLK), lambda j: (0, j)),
            pl.BlockSpec((H_BLK,), lambda j: (j,)),
            pl.BlockSpec((H_BLK, NUM_EXPERTS), lambda j: (j, 0)),
            pl.BlockSpec((NUM_EXPERTS,), lambda j: (0,)),
        ],
        out_specs=pl.BlockSpec((NUM_COMBOS, PACK_W), lambda j: (0, 0)),
        out_shape=jax.ShapeDtypeStruct((NUM_COMBOS, PACK_W), jnp.float32),
        scratch_shapes=[pltpu.VMEM((NUM_COMBOS, NUM_EXPERTS), jnp.float32)],
        compiler_params=pltpu.CompilerParams(
            dimension_semantics=("arbitrary",)),
    )(stage_table, view_table, W1, b1, W2, b2)


def _dispatch_body(table_hbm, sids_hbm, vids_hbm, out_hbm,
                   sid_v, vid_v, cidx_v, rows_v, sem, semw):
    wid = lax.axis_index("s") * SC_CORES + lax.axis_index("c")
    base = wid * B_PER_W
    pltpu.sync_copy(sids_hbm.at[pl.ds(base, B_PER_W)], sid_v)
    pltpu.sync_copy(vids_hbm.at[pl.ds(base, B_PER_W)], vid_v)
    for g in range(N_CHUNKS):
        for k in range(IDX_CHUNK // SC_LANES):
            off = g * IDX_CHUNK + k * SC_LANES
            s = sid_v[pl.ds(off, SC_LANES)]
            v = vid_v[pl.ds(off, SC_LANES)]
            cidx_v[g, pl.ds(k * SC_LANES, SC_LANES)] = s * NUM_VIEWS + v
    gathers = [
        pltpu.async_copy(table_hbm.at[cidx_v.at[g]], rows_v.at[g], sem)
        for g in range(N_CHUNKS)
    ]
    writes = []
    for g in range(N_CHUNKS):
        gathers[g].wait()
        writes.append(pltpu.async_copy(
            rows_v.at[g],
            out_hbm.at[pl.ds(base + g * IDX_CHUNK, IDX_CHUNK)], semw))
    for w in writes:
        w.wait()


def _dispatch(table, stage_ids, view_ids):
    mesh = plsc.VectorSubcoreMesh(core_axis_name="c", subcore_axis_name="s")
    run = functools.partial(
        pl.kernel,
        mesh=mesh,
        out_type=jax.ShapeDtypeStruct((B, PACK_W), jnp.float32),
        scratch_types=[
            pltpu.VMEM((B_PER_W,), jnp.int32),
            pltpu.VMEM((B_PER_W,), jnp.int32),
            pltpu.VMEM((N_CHUNKS, IDX_CHUNK), jnp.int32),
            pltpu.VMEM((N_CHUNKS, IDX_CHUNK, PACK_W), jnp.float32),
            pltpu.SemaphoreType.DMA,
            pltpu.SemaphoreType.DMA,
        ],
    )(_dispatch_body)
    return run(table, stage_ids, view_ids)


def kernel(stage_ids, view_ids, stage_table, view_table, W1, b1, W2, b2):
    stage_ids = stage_ids.astype(jnp.int32)
    view_ids = view_ids.astype(jnp.int32)
    table = _router_table(stage_table, view_table, W1, b1, W2, b2)
    packed = _dispatch(table, stage_ids, view_ids)
    expert_probs = packed[:, :NUM_EXPERTS]
    selected_expert = packed[:, NUM_EXPERTS].astype(jnp.int32)
    return (expert_probs, selected_expert)


# speedup vs baseline: 1.1013x; 1.0013x over previous
import functools

import jax
import jax.numpy as jnp
from jax import lax
from jax.experimental import pallas as pl
from jax.experimental.pallas import tpu as pltpu
from jax.experimental.pallas import tpu_sc as plsc

B = 16384
NUM_STAGES = 16
NUM_VIEWS = 8
NUM_COMBOS = NUM_STAGES * NUM_VIEWS
EMBED_DIM = 1024
HIDDEN_DIM = 4096
NUM_EXPERTS = 64
PACK_W = 48

H_BLK = 1024
N_HBLKS = HIDDEN_DIM // H_BLK

SC_CORES = 2
SC_SUBCORES = 16
SC_LANES = 16
NW = SC_CORES * SC_SUBCORES
B_PER_W = B // NW
IDX_CHUNK = 128
N_CHUNKS = B_PER_W // IDX_CHUNK


def _router_table_body(st_ref, vt_ref, w1_ref, b1_ref, w2_ref, b2_ref,
                       out_ref, acc_ref):
    j = pl.program_id(0)
    p1 = jnp.dot(st_ref[...], w1_ref[:EMBED_DIM, :],
                 preferred_element_type=jnp.float32)
    p2 = jnp.dot(vt_ref[...], w1_ref[EMBED_DIM:, :],
                 preferred_element_type=jnp.float32)
    p1r = jnp.broadcast_to(p1[:, None, :],
                           (NUM_STAGES, NUM_VIEWS, H_BLK))
    p1r = p1r.reshape(NUM_COMBOS, H_BLK)
    p2r = jnp.broadcast_to(p2[None, :, :],
                           (NUM_STAGES, NUM_VIEWS, H_BLK))
    p2r = p2r.reshape(NUM_COMBOS, H_BLK)
    h = jnp.maximum(p1r + p2r + b1_ref[...][None, :], 0.0)
    part = jnp.dot(h, w2_ref[...], preferred_element_type=jnp.float32)

    @pl.when(j == 0)
    def _init():
        acc_ref[...] = part

    @pl.when(j > 0)
    def _accum():
        acc_ref[...] = acc_ref[...] + part

    @pl.when(j == N_HBLKS - 1)
    def _finish():
        logits = acc_ref[...] + b2_ref[...][None, :]
        m = jnp.max(logits, axis=1, keepdims=True)
        e = jnp.exp(logits - m)
        probs = e / jnp.sum(e, axis=1, keepdims=True)
        col = lax.broadcasted_iota(jnp.int32, (NUM_COMBOS, NUM_EXPERTS), 1)
        pmax = jnp.max(probs, axis=1, keepdims=True)
        sel = jnp.min(jnp.where(probs == pmax, col, NUM_EXPERTS), axis=1)
        ub = jax.lax.bitcast_convert_type(probs, jnp.uint32)
        u = (ub + 0x7FFF + ((ub >> 16) & 1)) >> 16
        wp = (u[:, 32:] << 16) | u[:, :32]
        su = (jax.lax.bitcast_convert_type(sel.astype(jnp.float32),
                                           jnp.uint32) >> 16)
        ws = jnp.broadcast_to(((su << 16) | su)[:, None],
                              (NUM_COMBOS, PACK_W - 32))
        out_ref[...] = jax.lax.bitcast_convert_type(
            jnp.concatenate([wp, ws], axis=1), jnp.int32)


def _router_table(stage_table, view_table, W1, b1, W2, b2):
    return pl.pallas_call(
        _router_table_body,
        grid=(N_HBLKS,),
        in_specs=[
            pl.BlockSpec((NUM_STAGES, EMBED_DIM), lambda j: (0, 0)),
            pl.BlockSpec((NUM_VIEWS, EMBED_DIM), lambda j: (0, 0)),
            pl.BlockSpec((2 * EMBED_DIM, H_BLK), lambda j: (0, j)),
            pl.BlockSpec((H_BLK,), lambda j: (j,)),
            pl.BlockSpec((H_BLK, NUM_EXPERTS), lambda j: (j, 0)),
            pl.BlockSpec((NUM_EXPERTS,), lambda j: (0,)),
        ],
        out_specs=pl.BlockSpec((NUM_COMBOS, PACK_W), lambda j: (0, 0)),
        out_shape=jax.ShapeDtypeStruct((NUM_COMBOS, PACK_W), jnp.int32),
        scratch_shapes=[pltpu.VMEM((NUM_COMBOS, NUM_EXPERTS), jnp.float32)],
        compiler_params=pltpu.CompilerParams(
            dimension_semantics=("arbitrary",)),
    )(stage_table, view_table, W1, b1, W2, b2)


def _dispatch_body(table_hbm, sids_hbm, vids_hbm, out_hbm,
                   sid_v, vid_v, cidx_v, rows_v, sem, semw):
    wid = lax.axis_index("s") * SC_CORES + lax.axis_index("c")
    base = wid * B_PER_W
    pltpu.sync_copy(sids_hbm.at[pl.ds(base, B_PER_W)], sid_v)
    pltpu.sync_copy(vids_hbm.at[pl.ds(base, B_PER_W)], vid_v)
    for g in range(N_CHUNKS):
        for k in range(IDX_CHUNK // SC_LANES):
            off = g * IDX_CHUNK + k * SC_LANES
            s = sid_v[pl.ds(off, SC_LANES)]
            v = vid_v[pl.ds(off, SC_LANES)]
            cidx_v[g, pl.ds(k * SC_LANES, SC_LANES)] = s * NUM_VIEWS + v
    gathers = [
        pltpu.async_copy(table_hbm.at[cidx_v.at[g]], rows_v.at[g], sem)
        for g in range(N_CHUNKS)
    ]
    writes = []
    for g in range(N_CHUNKS):
        gathers[g].wait()
        writes.append(pltpu.async_copy(
            rows_v.at[g],
            out_hbm.at[pl.ds(base + g * IDX_CHUNK, IDX_CHUNK)], semw))
    for w in writes:
        w.wait()


def _dispatch(table, stage_ids, view_ids):
    mesh = plsc.VectorSubcoreMesh(core_axis_name="c", subcore_axis_name="s")
    run = functools.partial(
        pl.kernel,
        mesh=mesh,
        out_type=jax.ShapeDtypeStruct((B, PACK_W), jnp.int32),
        scratch_types=[
            pltpu.VMEM((B_PER_W,), jnp.int32),
            pltpu.VMEM((B_PER_W,), jnp.int32),
            pltpu.VMEM((N_CHUNKS, IDX_CHUNK), jnp.int32),
            pltpu.VMEM((N_CHUNKS, IDX_CHUNK, PACK_W), jnp.int32),
            pltpu.SemaphoreType.DMA,
            pltpu.SemaphoreType.DMA,
        ],
        compiler_params=pltpu.CompilerParams(use_tc_tiling_on_sc=False),
    )(_dispatch_body)
    return run(table, stage_ids, view_ids)


def kernel(stage_ids, view_ids, stage_table, view_table, W1, b1, W2, b2):
    stage_ids = stage_ids.astype(jnp.int32)
    view_ids = view_ids.astype(jnp.int32)
    table = _router_table(stage_table, view_table, W1, b1, W2, b2)
    packed = _dispatch(table, stage_ids, view_ids)
    pb = jax.lax.bitcast_convert_type(packed, jnp.bfloat16)
    expert_probs = jnp.concatenate(
        [pb[:, :32, 0], pb[:, :32, 1]], axis=-1).astype(jnp.float32)
    selected_expert = pb[:, 32, 0].astype(jnp.int32)
    return (expert_probs, selected_expert)
